# baseline (speedup 1.0000x reference)
import numpy as np
import jax
import jax.numpy as jnp
from jax.experimental import pallas as pl
from jax.experimental.pallas import tpu as pltpu


def _lane_matrix(w: int) -> np.ndarray:
    m = np.zeros((w, 2 * w), np.float32)
    j = np.arange(w)
    m[np.maximum(j - 1, 0), 2 * j] += 0.25
    m[j, 2 * j] += 0.75
    m[j, 2 * j + 1] += 0.75
    m[np.minimum(j + 1, w - 1), 2 * j + 1] += 0.25
    return m


def _up2x_kernel(x_ref, bt_ref, o_ref):
    t, h, w = x_ref.shape
    w2 = bt_ref.shape[1]
    y = jnp.dot(x_ref[...].reshape(t * h, w), bt_ref[...],
                preferred_element_type=jnp.float32).reshape(t, h, w2)
    y_up = jnp.concatenate([y[:, :1], y[:, :-1]], axis=1)
    y_dn = jnp.concatenate([y[:, 1:], y[:, -1:]], axis=1)
    even = 0.75 * y + 0.25 * y_up
    odd = 0.75 * y + 0.25 * y_dn
    o_ref[...] = jnp.concatenate([even, odd], axis=-1)


def kernel(x):
    n, c, h, w = map(int, x.shape)
    nc = n * c
    tile = 64
    while nc % tile:
        tile -= 1
    bt = jnp.asarray(_lane_matrix(w))
    out = pl.pallas_call(
        _up2x_kernel,
        out_shape=jax.ShapeDtypeStruct((nc, h, 4 * w), x.dtype),
        grid=(nc // tile,),
        in_specs=[pl.BlockSpec((tile, h, w), lambda i: (i, 0, 0)),
                  pl.BlockSpec((w, 2 * w), lambda i: (0, 0))],
        out_specs=pl.BlockSpec((tile, h, 4 * w), lambda i: (i, 0, 0)),
        compiler_params=pltpu.CompilerParams(
            dimension_semantics=("parallel",),
        ),
    )(x.reshape(nc, h, w), bt)
    return out.reshape(n, c, 2 * h, 2 * w)


# speedup vs baseline: 1.2979x; 1.2979x over previous
import numpy as np
import jax
import jax.numpy as jnp
from jax.experimental import pallas as pl
from jax.experimental.pallas import tpu as pltpu


def _lane_matrix(w: int) -> np.ndarray:
    m = np.zeros((w, 2 * w), np.float32)
    j = np.arange(w)
    m[np.maximum(j - 1, 0), 2 * j] += 0.25
    m[j, 2 * j] += 0.75
    m[j, 2 * j + 1] += 0.75
    m[np.minimum(j + 1, w - 1), 2 * j + 1] += 0.25
    return m


def _up2x_kernel(x_ref, bt_ref, o_ref):
    t, h, w = x_ref.shape
    w2 = bt_ref.shape[1]
    y = jnp.dot(x_ref[...].reshape(t * h, w), bt_ref[...],
                preferred_element_type=jnp.float32).reshape(t, h, w2)
    y_up = jnp.concatenate([y[:, :1], y[:, :-1]], axis=1)
    y_dn = jnp.concatenate([y[:, 1:], y[:, -1:]], axis=1)
    even = 0.75 * y + 0.25 * y_up
    odd = 0.75 * y + 0.25 * y_dn
    o_ref[...] = jnp.stack([even, odd], axis=2).reshape(t, 2 * h, w2)


def kernel(x):
    n, c, h, w = map(int, x.shape)
    nc = n * c
    tile = 128
    while nc % tile:
        tile -= 1
    bt = jnp.asarray(_lane_matrix(w))
    out = pl.pallas_call(
        _up2x_kernel,
        out_shape=jax.ShapeDtypeStruct((nc, 2 * h, 2 * w), x.dtype),
        grid=(nc // tile,),
        in_specs=[pl.BlockSpec((tile, h, w), lambda i: (i, 0, 0)),
                  pl.BlockSpec((w, 2 * w), lambda i: (0, 0))],
        out_specs=pl.BlockSpec((tile, 2 * h, 2 * w), lambda i: (i, 0, 0)),
        compiler_params=pltpu.CompilerParams(
            dimension_semantics=("parallel",),
            vmem_limit_bytes=56 * 1024 * 1024,
        ),
    )(x.reshape(nc, h, w), bt)
    return out.reshape(n, c, 2 * h, 2 * w)


# speedup vs baseline: 1.9586x; 1.5091x over previous
import numpy as np
import jax
import jax.numpy as jnp
from jax.experimental import pallas as pl
from jax.experimental.pallas import tpu as pltpu


def _lane_matrix(w: int) -> np.ndarray:
    m = np.zeros((w, 2 * w), np.float32)
    j = np.arange(w)
    m[np.maximum(j - 1, 0), 2 * j] += 0.25
    m[j, 2 * j] += 0.75
    m[j, 2 * j + 1] += 0.75
    m[np.minimum(j + 1, w - 1), 2 * j + 1] += 0.25
    return m


def _up2x_kernel(x_ref, bt_ref, o_ref):
    _, t, h, w = x_ref.shape
    w2 = bt_ref.shape[1]
    y = jnp.dot(x_ref[0].reshape(t * h, w), bt_ref[...],
                preferred_element_type=jnp.float32).reshape(t, h, w2)
    y_up = jnp.concatenate([y[:, :1], y[:, :-1]], axis=1)
    y_dn = jnp.concatenate([y[:, 1:], y[:, -1:]], axis=1)
    o_ref[:, ::2, :] = 0.75 * y + 0.25 * y_up
    o_ref[:, 1::2, :] = 0.75 * y + 0.25 * y_dn


def kernel(x):
    n, c, h, w = map(int, x.shape)
    ctile = 128
    bt = jnp.asarray(_lane_matrix(w))
    out = pl.pallas_call(
        _up2x_kernel,
        out_shape=jax.ShapeDtypeStruct((n * c, 2 * h, 2 * w), x.dtype),
        grid=(n, c // ctile),
        in_specs=[pl.BlockSpec((1, ctile, h, w), lambda i, j: (i, j, 0, 0)),
                  pl.BlockSpec((w, 2 * w), lambda i, j: (0, 0))],
        out_specs=pl.BlockSpec((ctile, 2 * h, 2 * w),
                               lambda i, j, _c=c // ctile: (i * _c + j, 0, 0)),
        compiler_params=pltpu.CompilerParams(
            dimension_semantics=("parallel", "parallel"),
            vmem_limit_bytes=56 * 1024 * 1024,
        ),
    )(x, bt)
    return out.reshape(n, c, 2 * h, 2 * w)


# speedup vs baseline: 2.7987x; 1.4289x over previous
import numpy as np
import jax
import jax.numpy as jnp
from jax.experimental import pallas as pl
from jax.experimental.pallas import tpu as pltpu


def _lane_matrix(w: int) -> np.ndarray:
    m = np.zeros((w, 2 * w), np.float32)
    j = np.arange(w)
    m[np.maximum(j - 1, 0), 2 * j] += 0.25
    m[j, 2 * j] += 0.75
    m[j, 2 * j + 1] += 0.75
    m[np.minimum(j + 1, w - 1), 2 * j + 1] += 0.25
    return m


def _pair_matrix(w: int) -> np.ndarray:
    b = _lane_matrix(w)
    m = np.zeros((2 * w, 4 * w), np.float32)
    m[:w, : 2 * w] = b
    m[w:, 2 * w:] = b
    return m


def _up2x_kernel(x_ref, bt2_ref, o_ref):
    _, t, hp, wp = x_ref.shape
    w2 = bt2_ref.shape[1] // 2
    y = jnp.dot(x_ref[0].reshape(t * hp, wp), bt2_ref[...],
                preferred_element_type=jnp.float32).reshape(t, hp, 2 * w2)
    e = y[:, :, :w2]
    o = y[:, :, w2:]
    o_up = jnp.concatenate([e[:, :1], o[:, :-1]], axis=1)
    e_dn = jnp.concatenate([e[:, 1:], o[:, -1:]], axis=1)
    o_ref[:, 0::4, :] = 0.25 * o_up + 0.75 * e
    o_ref[:, 1::4, :] = 0.75 * e + 0.25 * o
    o_ref[:, 2::4, :] = 0.25 * e + 0.75 * o
    o_ref[:, 3::4, :] = 0.75 * o + 0.25 * e_dn


def kernel(x):
    n, c, h, w = map(int, x.shape)
    ctile = 128
    bt2 = jnp.asarray(_pair_matrix(w))
    out = pl.pallas_call(
        _up2x_kernel,
        out_shape=jax.ShapeDtypeStruct((n * c, 2 * h, 2 * w), x.dtype),
        grid=(n, c // ctile),
        in_specs=[pl.BlockSpec((1, ctile, h // 2, 2 * w),
                               lambda i, j: (i, j, 0, 0)),
                  pl.BlockSpec((2 * w, 4 * w), lambda i, j: (0, 0))],
        out_specs=pl.BlockSpec((ctile, 2 * h, 2 * w),
                               lambda i, j, _c=c // ctile: (i * _c + j, 0, 0)),
        compiler_params=pltpu.CompilerParams(
            dimension_semantics=("parallel", "parallel"),
            vmem_limit_bytes=56 * 1024 * 1024,
        ),
    )(x.reshape(n, c, h // 2, 2 * w), bt2)
    return out.reshape(n, c, 2 * h, 2 * w)


# speedup vs baseline: 2.8587x; 1.0215x over previous
import numpy as np
import jax
import jax.numpy as jnp
from jax.experimental import pallas as pl
from jax.experimental.pallas import tpu as pltpu


def _lane_matrix(w: int) -> np.ndarray:
    m = np.zeros((w, 2 * w), np.float32)
    j = np.arange(w)
    m[np.maximum(j - 1, 0), 2 * j] += 0.25
    m[j, 2 * j] += 0.75
    m[j, 2 * j + 1] += 0.75
    m[np.minimum(j + 1, w - 1), 2 * j + 1] += 0.25
    return m


def _pair_matrix(w: int) -> np.ndarray:
    b = _lane_matrix(w)
    m = np.zeros((2 * w, 4 * w), np.float32)
    m[:w, : 2 * w] = b
    m[w:, 2 * w:] = b
    return m


def _up2x_kernel(x_ref, bt2_ref, o_ref):
    _, t, hw = x_ref.shape
    wp = bt2_ref.shape[0]
    w2 = bt2_ref.shape[1] // 2
    hp = hw // wp
    y = jnp.dot(x_ref[0].reshape(t * hp, wp), bt2_ref[...],
                preferred_element_type=jnp.float32).reshape(t, hp, 2 * w2)
    e = y[:, :, :w2]
    o = y[:, :, w2:]
    o_up = jnp.concatenate([e[:, :1], o[:, :-1]], axis=1)
    e_dn = jnp.concatenate([e[:, 1:], o[:, -1:]], axis=1)
    o_ref[:, 0::4, :] = 0.25 * o_up + 0.75 * e
    o_ref[:, 1::4, :] = 0.75 * e + 0.25 * o
    o_ref[:, 2::4, :] = 0.25 * e + 0.75 * o
    o_ref[:, 3::4, :] = 0.75 * o + 0.25 * e_dn


def kernel(x):
    n, c, h, w = map(int, x.shape)
    ctile = 128
    bt2 = jnp.asarray(_pair_matrix(w))
    out = pl.pallas_call(
        _up2x_kernel,
        out_shape=jax.ShapeDtypeStruct((n * c, 2 * h, 2 * w), x.dtype),
        grid=(n, c // ctile),
        in_specs=[pl.BlockSpec((1, ctile, h * w),
                               lambda i, j: (i, j, 0)),
                  pl.BlockSpec((2 * w, 4 * w), lambda i, j: (0, 0))],
        out_specs=pl.BlockSpec((ctile, 2 * h, 2 * w),
                               lambda i, j, _c=c // ctile: (i * _c + j, 0, 0)),
        compiler_params=pltpu.CompilerParams(
            dimension_semantics=("parallel", "parallel"),
            vmem_limit_bytes=56 * 1024 * 1024,
        ),
    )(x.reshape(n, c, h * w), bt2)
    return out.reshape(n, c, 2 * h, 2 * w)


# speedup vs baseline: 4.4276x; 1.5488x over previous
import numpy as np
import jax
import jax.numpy as jnp
from jax.experimental import pallas as pl
from jax.experimental.pallas import tpu as pltpu


def _lane_matrix(w: int) -> np.ndarray:
    m = np.zeros((w, 2 * w), np.float32)
    j = np.arange(w)
    m[np.maximum(j - 1, 0), 2 * j] += 0.25
    m[j, 2 * j] += 0.75
    m[j, 2 * j + 1] += 0.75
    m[np.minimum(j + 1, w - 1), 2 * j + 1] += 0.25
    return m


def _pair_matrix(w: int) -> np.ndarray:
    b = _lane_matrix(w)
    m = np.zeros((2 * w, 4 * w), np.float32)
    m[:w, : 2 * w] = b
    m[w:, 2 * w:] = b
    return m


def _up2x_kernel(x_ref, bt2_ref, o_ref):
    _, hw, t = x_ref.shape
    wp = bt2_ref.shape[0]
    w2 = bt2_ref.shape[1] // 2
    hp = hw // wp
    xt = x_ref[0].T
    y = jnp.dot(xt.reshape(t * hp, wp), bt2_ref[...],
                preferred_element_type=jnp.float32).reshape(t, hp, 2 * w2)
    e = y[:, :, :w2]
    o = y[:, :, w2:]
    o_up = jnp.concatenate([e[:, :1], o[:, :-1]], axis=1)
    e_dn = jnp.concatenate([e[:, 1:], o[:, -1:]], axis=1)
    o_ref[:, 0::4, :] = 0.25 * o_up + 0.75 * e
    o_ref[:, 1::4, :] = 0.75 * e + 0.25 * o
    o_ref[:, 2::4, :] = 0.25 * e + 0.75 * o
    o_ref[:, 3::4, :] = 0.75 * o + 0.25 * e_dn


def kernel(x):
    n, c, h, w = map(int, x.shape)
    ctile = 128
    bt2 = jnp.asarray(_pair_matrix(w))
    out = pl.pallas_call(
        _up2x_kernel,
        out_shape=jax.ShapeDtypeStruct((n * c, 2 * h, 2 * w), x.dtype),
        grid=(n, c // ctile),
        in_specs=[pl.BlockSpec((1, h * w, ctile),
                               lambda i, j: (i, 0, j)),
                  pl.BlockSpec((2 * w, 4 * w), lambda i, j: (0, 0))],
        out_specs=pl.BlockSpec((ctile, 2 * h, 2 * w),
                               lambda i, j, _c=c // ctile: (i * _c + j, 0, 0)),
        compiler_params=pltpu.CompilerParams(
            dimension_semantics=("parallel", "parallel"),
            vmem_limit_bytes=56 * 1024 * 1024,
        ),
    )(jnp.transpose(x.reshape(n, c, h * w), (0, 2, 1)), bt2)
    return out.reshape(n, c, 2 * h, 2 * w)


# speedup vs baseline: 4.5176x; 1.0203x over previous
import numpy as np
import jax
import jax.numpy as jnp
from jax.experimental import pallas as pl
from jax.experimental.pallas import tpu as pltpu


def _lane_matrix(w: int) -> np.ndarray:
    m = np.zeros((w, 2 * w), np.float32)
    j = np.arange(w)
    m[np.maximum(j - 1, 0), 2 * j] += 0.25
    m[j, 2 * j] += 0.75
    m[j, 2 * j + 1] += 0.75
    m[np.minimum(j + 1, w - 1), 2 * j + 1] += 0.25
    return m


def _pair_matrix(w: int) -> np.ndarray:
    b = _lane_matrix(w)
    m = np.zeros((2 * w, 4 * w), np.float32)
    m[:w, : 2 * w] = b
    m[w:, 2 * w:] = b
    return m


def _up2x_kernel(x_ref, bt2_ref, o_ref):
    _, hw, t = x_ref.shape
    wp = bt2_ref.shape[0]
    w2 = bt2_ref.shape[1] // 2
    hp = hw // wp
    xt = x_ref[0].T
    y = jnp.dot(xt.reshape(t * hp, wp), bt2_ref[...],
                preferred_element_type=jnp.float32).reshape(t, hp, 2 * w2)
    e = y[:, :, :w2]
    o = y[:, :, w2:]
    o_up = jnp.concatenate([e[:, :1], o[:, :-1]], axis=1)
    e_dn = jnp.concatenate([e[:, 1:], o[:, -1:]], axis=1)
    o_ref[:, 0::4, :] = 0.25 * o_up + 0.75 * e
    o_ref[:, 1::4, :] = 0.75 * e + 0.25 * o
    o_ref[:, 2::4, :] = 0.25 * e + 0.75 * o
    o_ref[:, 3::4, :] = 0.75 * o + 0.25 * e_dn


def kernel(x):
    n, c, h, w = map(int, x.shape)
    ctile = 256
    bt2 = jnp.asarray(_pair_matrix(w))
    out = pl.pallas_call(
        _up2x_kernel,
        out_shape=jax.ShapeDtypeStruct((n * c, 2 * h, 2 * w), x.dtype),
        grid=(n, c // ctile),
        in_specs=[pl.BlockSpec((1, h * w, ctile),
                               lambda i, j: (i, 0, j)),
                  pl.BlockSpec((2 * w, 4 * w), lambda i, j: (0, 0))],
        out_specs=pl.BlockSpec((ctile, 2 * h, 2 * w),
                               lambda i, j, _c=c // ctile: (i * _c + j, 0, 0)),
        compiler_params=pltpu.CompilerParams(
            dimension_semantics=("parallel", "parallel"),
            vmem_limit_bytes=56 * 1024 * 1024,
        ),
    )(jnp.transpose(x.reshape(n, c, h * w), (0, 2, 1)), bt2)
    return out.reshape(n, c, 2 * h, 2 * w)
